# baseline (speedup 1.0000x reference)
import jax
import jax.numpy as jnp
from jax.experimental import pallas as pl
from jax.experimental.pallas import tpu as pltpu

ALPHA = 1.702
LIMIT = 7.0

BM = 256


def _dot(a, b, prec=None):
    return jax.lax.dot_general(
        a, b, (((1,), (0,)), ((), ())),
        precision=prec, preferred_element_type=jnp.float32)


def _moe_kernel(x32_ref, xbf_ref, rw_ref, rb_ref, gup_ref, gub_ref,
                dw_ref, db_ref, out_ref, acc_ref, scores_ref):
    e = pl.program_id(0)
    m = pl.program_id(1)
    n_e = pl.num_programs(0)
    E = rw_ref.shape[1]
    row0 = m * BM

    @pl.when(e == 0)
    def _router():
        xb = x32_ref[...]
        l = _dot(xb, rw_ref[...]) + rb_ref[...]
        m1 = jnp.max(l, axis=1, keepdims=True)
        tri = (jax.lax.broadcasted_iota(jnp.int32, (E, E), 0)
               <= jax.lax.broadcasted_iota(jnp.int32, (E, E), 1)).astype(jnp.float32)
        is1 = (l == m1).astype(jnp.float32)
        r1 = _dot(is1, tri, prec=jax.lax.Precision.HIGHEST)
        sel1 = is1 * (r1 == 1.0).astype(jnp.float32)
        l2 = jnp.where(sel1 > 0.0, -jnp.inf, l)
        m2 = jnp.max(l2, axis=1, keepdims=True)
        is2 = (l2 == m2).astype(jnp.float32)
        r2 = _dot(is2, tri, prec=jax.lax.Precision.HIGHEST)
        sel2 = is2 * (r2 == 1.0).astype(jnp.float32)
        e2 = jnp.exp(m2 - m1)
        p1 = 1.0 / (1.0 + e2)
        p2 = e2 / (1.0 + e2)
        scores_ref[pl.ds(row0, BM), :] = p1 * sel1 + p2 * sel2

    xb = xbf_ref[...]
    wgu = gup_ref[0].astype(jnp.bfloat16)
    gu = _dot(xb, wgu) + gub_ref[0]
    lane = jax.lax.broadcasted_iota(jnp.int32, gu.shape, 1)
    even = (lane % 2) == 0
    g = jnp.minimum(gu, LIMIT)
    glu = g * jax.nn.sigmoid(g * ALPHA)
    up1 = jnp.clip(gu, -LIMIT, LIMIT) + 1.0
    act_full = jnp.where(even, glu, up1)
    act_pair = act_full * pltpu.roll(act_full, act_full.shape[1] - 1, 1)
    act = act_pair.reshape(act_pair.shape[0], act_pair.shape[1] // 2, 2)[:, :, 0]
    dwn = _dot(act.astype(jnp.bfloat16), dw_ref[0].astype(jnp.bfloat16)) + db_ref[0]
    sc = scores_ref[pl.ds(row0, BM), :]
    sel = (jax.lax.broadcasted_iota(jnp.int32, sc.shape, 1) == e).astype(jnp.float32)
    w_col = jnp.sum(sc * sel, axis=1, keepdims=True)
    contrib = dwn * w_col

    @pl.when(e == 0)
    def _init():
        acc_ref[pl.ds(row0, BM), :] = contrib

    @pl.when(e > 0)
    def _acc():
        acc_ref[pl.ds(row0, BM), :] += contrib

    @pl.when(e == n_e - 1)
    def _emit():
        out_ref[...] = acc_ref[pl.ds(row0, BM), :]


def kernel(hidden_states, router_w, router_b, gate_up_proj, gate_up_bias,
           down_proj, down_bias):
    B, S, H = hidden_states.shape
    E, _, F2 = gate_up_proj.shape
    I = F2 // 2
    T = B * S
    NM = T // BM

    x32 = hidden_states.reshape(T, H)
    xbf = x32.astype(jnp.bfloat16)
    rb2 = router_b.reshape(1, E)
    gub = gate_up_bias.reshape(E, 1, F2)
    db3 = down_bias.reshape(E, 1, H)

    grid = (E, NM)
    out = pl.pallas_call(
        _moe_kernel,
        grid=grid,
        in_specs=[
            pl.BlockSpec((BM, H), lambda e, m: (jnp.where(e == 0, m, 0), 0)),
            pl.BlockSpec((BM, H), lambda e, m: (m, 0)),
            pl.BlockSpec((H, E), lambda e, m: (0, 0)),
            pl.BlockSpec((1, E), lambda e, m: (0, 0)),
            pl.BlockSpec((1, H, F2), lambda e, m: (e, 0, 0)),
            pl.BlockSpec((1, 1, F2), lambda e, m: (e, 0, 0)),
            pl.BlockSpec((1, I, H), lambda e, m: (e, 0, 0)),
            pl.BlockSpec((1, 1, H), lambda e, m: (e, 0, 0)),
        ],
        out_specs=pl.BlockSpec(
            (BM, H), lambda e, m: (jnp.where(e == E - 1, m, 0), 0)),
        out_shape=jax.ShapeDtypeStruct((T, H), jnp.float32),
        scratch_shapes=[
            pltpu.VMEM((T, H), jnp.float32),
            pltpu.VMEM((T, E), jnp.float32),
        ],
        compiler_params=pltpu.CompilerParams(
            dimension_semantics=("arbitrary", "arbitrary"),
            vmem_limit_bytes=100 * 1024 * 1024,
        ),
    )(x32, xbf, router_w, rb2, gate_up_proj, gub, down_proj, db3)
    return out.reshape(B, S, H)


# speedup vs baseline: 22.0582x; 22.0582x over previous
import jax
import jax.numpy as jnp
from jax.experimental import pallas as pl
from jax.experimental.pallas import tpu as pltpu

ALPHA = 1.702
LIMIT = 7.0

BM = 256


def _dot(a, b, prec=None):
    return jax.lax.dot_general(
        a, b, (((1,), (0,)), ((), ())),
        precision=prec, preferred_element_type=jnp.float32)


def _moe_kernel(x32_ref, xbf_ref, rw_ref, rb_ref, gup_ref, gbe_ref, gbo_ref,
                dw_ref, db_ref, out_ref, acc_ref, scores_ref,
                sg_ref, su_ref, wg_ref, wu_ref):
    e = pl.program_id(0)
    m = pl.program_id(1)
    n_e = pl.num_programs(0)
    E = rw_ref.shape[1]
    row0 = m * BM

    @pl.when(e == 0)
    def _router():
        xb = x32_ref[...]
        l = _dot(xb, rw_ref[...]) + rb_ref[...]
        m1 = jnp.max(l, axis=1, keepdims=True)
        tri = (jax.lax.broadcasted_iota(jnp.int32, (E, E), 0)
               <= jax.lax.broadcasted_iota(jnp.int32, (E, E), 1)).astype(jnp.float32)
        is1 = (l == m1).astype(jnp.float32)
        r1 = _dot(is1, tri, prec=jax.lax.Precision.HIGHEST)
        sel1 = is1 * (r1 == 1.0).astype(jnp.float32)
        l2 = jnp.where(sel1 > 0.0, -jnp.inf, l)
        m2 = jnp.max(l2, axis=1, keepdims=True)
        is2 = (l2 == m2).astype(jnp.float32)
        r2 = _dot(is2, tri, prec=jax.lax.Precision.HIGHEST)
        sel2 = is2 * (r2 == 1.0).astype(jnp.float32)
        e2 = jnp.exp(m2 - m1)
        p1 = 1.0 / (1.0 + e2)
        p2 = e2 / (1.0 + e2)
        scores_ref[pl.ds(row0, BM), :] = p1 * sel1 + p2 * sel2

    @pl.when((e == 0) & (m == 0))
    def _build_sel():
        F2, I = sg_ref.shape[0], sg_ref.shape[1]
        r = jax.lax.broadcasted_iota(jnp.int32, (F2, I), 0)
        col = jax.lax.broadcasted_iota(jnp.int32, (F2, I), 1)
        sg_ref[...] = (r == 2 * col).astype(jnp.bfloat16)
        su_ref[...] = (r == 2 * col + 1).astype(jnp.bfloat16)

    @pl.when(m == 0)
    def _split_weights():
        wgu = gup_ref[0].astype(jnp.bfloat16)
        wg_ref[...] = _dot(wgu, sg_ref[...]).astype(jnp.bfloat16)
        wu_ref[...] = _dot(wgu, su_ref[...]).astype(jnp.bfloat16)

    xb = xbf_ref[...]
    g = _dot(xb, wg_ref[...]) + gbe_ref[0]
    u = _dot(xb, wu_ref[...]) + gbo_ref[0]
    g = jnp.minimum(g, LIMIT)
    glu = g * jax.nn.sigmoid(g * ALPHA)
    up1 = jnp.clip(u, -LIMIT, LIMIT) + 1.0
    act = up1 * glu
    dwn = _dot(act.astype(jnp.bfloat16), dw_ref[0].astype(jnp.bfloat16)) + db_ref[0]
    sc = scores_ref[pl.ds(row0, BM), :]
    sel = (jax.lax.broadcasted_iota(jnp.int32, sc.shape, 1) == e).astype(jnp.float32)
    w_col = jnp.sum(sc * sel, axis=1, keepdims=True)
    contrib = dwn * w_col

    @pl.when(e == 0)
    def _init():
        acc_ref[pl.ds(row0, BM), :] = contrib

    @pl.when(e > 0)
    def _acc():
        acc_ref[pl.ds(row0, BM), :] += contrib

    @pl.when(e == n_e - 1)
    def _emit():
        out_ref[...] = acc_ref[pl.ds(row0, BM), :]


def kernel(hidden_states, router_w, router_b, gate_up_proj, gate_up_bias,
           down_proj, down_bias):
    B, S, H = hidden_states.shape
    E, _, F2 = gate_up_proj.shape
    I = F2 // 2
    T = B * S
    NM = T // BM

    x32 = hidden_states.reshape(T, H)
    xbf = x32.astype(jnp.bfloat16)
    rb2 = router_b.reshape(1, E)
    gbe = gate_up_bias[:, 0::2].reshape(E, 1, I)
    gbo = gate_up_bias[:, 1::2].reshape(E, 1, I)
    db3 = down_bias.reshape(E, 1, H)

    grid = (E, NM)
    out = pl.pallas_call(
        _moe_kernel,
        grid=grid,
        in_specs=[
            pl.BlockSpec((BM, H), lambda e, m: (jnp.where(e == 0, m, 0), 0)),
            pl.BlockSpec((BM, H), lambda e, m: (m, 0)),
            pl.BlockSpec((H, E), lambda e, m: (0, 0)),
            pl.BlockSpec((1, E), lambda e, m: (0, 0)),
            pl.BlockSpec((1, H, F2), lambda e, m: (e, 0, 0)),
            pl.BlockSpec((1, 1, I), lambda e, m: (e, 0, 0)),
            pl.BlockSpec((1, 1, I), lambda e, m: (e, 0, 0)),
            pl.BlockSpec((1, I, H), lambda e, m: (e, 0, 0)),
            pl.BlockSpec((1, 1, H), lambda e, m: (e, 0, 0)),
        ],
        out_specs=pl.BlockSpec(
            (BM, H), lambda e, m: (jnp.where(e == E - 1, m, 0), 0)),
        out_shape=jax.ShapeDtypeStruct((T, H), jnp.float32),
        scratch_shapes=[
            pltpu.VMEM((T, H), jnp.float32),
            pltpu.VMEM((T, E), jnp.float32),
            pltpu.VMEM((F2, I), jnp.bfloat16),
            pltpu.VMEM((F2, I), jnp.bfloat16),
            pltpu.VMEM((H, I), jnp.bfloat16),
            pltpu.VMEM((H, I), jnp.bfloat16),
        ],
        compiler_params=pltpu.CompilerParams(
            dimension_semantics=("arbitrary", "arbitrary"),
            vmem_limit_bytes=100 * 1024 * 1024,
        ),
    )(x32, xbf, router_w, rb2, gate_up_proj, gbe, gbo, down_proj, db3)
    return out.reshape(B, S, H)
